# baseline (speedup 1.0000x reference)
import functools

import jax
import jax.numpy as jnp
from jax import lax
from jax.experimental import pallas as pl
from jax.experimental.pallas import tpu as pltpu
from jax.experimental.pallas import tpu_sc as plsc

_NC = 2
_NS = 16
_IW = 128
_R = 8


def kernel(edge_idxs, table):
    n = edge_idxs.shape[0]
    d = table.shape[1]
    nw = _NC * _NS
    chunk = _IW * _R
    n_chunks = n // chunk
    max_iters = (n_chunks + nw - 1) // nw
    idx2d = edge_idxs.reshape(n // _IW, _IW)
    mesh = plsc.VectorSubcoreMesh(core_axis_name="c", subcore_axis_name="s")

    @functools.partial(
        pl.kernel,
        mesh=mesh,
        out_type=jax.ShapeDtypeStruct((n, d), table.dtype),
        scratch_types=[
            pltpu.VMEM((_R, _IW), jnp.int32),
            pltpu.VMEM((chunk, d), jnp.float32),
            pltpu.SemaphoreType.DMA,
        ],
        compiler_params=pltpu.CompilerParams(use_tc_tiling_on_sc=False),
    )
    def gather_kernel(table_hbm, idx_hbm, out_hbm, idx_v, rows_v, sem):
        wid = lax.axis_index("s") * _NC + lax.axis_index("c")

        @pl.loop(0, max_iters)
        def _(i):
            c = wid + i * nw

            @pl.when(c < n_chunks)
            def _():
                pltpu.sync_copy(idx_hbm.at[pl.ds(c * _R, _R)], idx_v)
                copies = []
                for j in range(_R):
                    copies.append(pltpu.async_copy(
                        table_hbm.at[idx_v.at[j]],
                        rows_v.at[pl.ds(j * _IW, _IW)],
                        sem,
                    ))
                for cp in copies:
                    cp.wait()
                pltpu.sync_copy(rows_v, out_hbm.at[pl.ds(c * chunk, chunk)])

    return gather_kernel(table, idx2d)


# speedup vs baseline: 1.0670x; 1.0670x over previous
import functools

import jax
import jax.numpy as jnp
from jax import lax
from jax.experimental import pallas as pl
from jax.experimental.pallas import tpu as pltpu
from jax.experimental.pallas import tpu_sc as plsc

_NC = 2
_NS = 16
_L = 128
_U = 8


def _views(n, d):
    nb = n // _L
    na = d // 8
    return nb, na


def kernel(edge_idxs, table):
    n, d = table.shape
    nb, na = _views(n, d)
    nw = _NC * _NS
    n_units = nb // _U
    max_iters = (n_units + nw - 1) // nw
    mesh = plsc.VectorSubcoreMesh(core_axis_name="c", subcore_axis_name="s")
    cp = pltpu.CompilerParams(use_tc_tiling_on_sc=False,
                              needs_layout_passes=False)

    q = table.reshape(nb, _L, na, 8).transpose(2, 0, 3, 1)
    idx2d = edge_idxs.reshape(nb, _L)

    @functools.partial(
        pl.kernel,
        mesh=mesh,
        out_type=jax.ShapeDtypeStruct((n, d), table.dtype),
        scratch_types=[
            pltpu.VMEM((na, _U, 8, _L), jnp.float32),
            pltpu.VMEM((_U * _L, d), jnp.float32),
            pltpu.SemaphoreType.DMA,
        ],
        compiler_params=cp,
    )
    def relayout(q_hbm, rm_hbm, tin_v, rbuf_v, sem):
        wid = lax.axis_index("s") * _NC + lax.axis_index("c")
        ja = jnp.arange(16, dtype=jnp.int32) // 8
        jr = jnp.arange(16, dtype=jnp.int32) % 8

        @pl.loop(0, max_iters)
        def _(i):
            u = wid + i * nw

            @pl.when(u < n_units)
            def _():
                c0 = pltpu.async_copy(q_hbm.at[0, pl.ds(u * _U, _U)],
                                      tin_v.at[0], sem)
                c1 = pltpu.async_copy(q_hbm.at[1, pl.ds(u * _U, _U)],
                                      tin_v.at[1], sem)
                c0.wait()
                c1.wait()

                @pl.loop(0, _U)
                def _(b):
                    jb = jnp.broadcast_to(b, (16,)).astype(jnp.int32)

                    @pl.loop(0, _L)
                    def _(l):
                        jl = jnp.broadcast_to(l, (16,)).astype(jnp.int32)
                        vals = plsc.load_gather(tin_v, [ja, jb, jr, jl])
                        rbuf_v[b * _L + l] = vals

                pltpu.sync_copy(rbuf_v, rm_hbm.at[pl.ds(u * _U * _L, _U * _L)])

    @functools.partial(
        pl.kernel,
        mesh=mesh,
        out_type=jax.ShapeDtypeStruct((na, nb, 8, _L), jnp.float32),
        scratch_types=[
            pltpu.VMEM((_U, _L), jnp.int32),
            pltpu.VMEM((_U * _L, d), jnp.float32),
            pltpu.VMEM((na, _U, 8, _L), jnp.float32),
            pltpu.SemaphoreType.DMA,
        ],
        compiler_params=cp,
    )
    def gather(rm_hbm, idx_hbm, out_hbm, idx_v, rows_v, tiles_v, sem):
        wid = lax.axis_index("s") * _NC + lax.axis_index("c")
        ja = jnp.arange(16, dtype=jnp.int32) // 8
        jr = jnp.arange(16, dtype=jnp.int32) % 8

        @pl.loop(0, max_iters)
        def _(i):
            u = wid + i * nw

            @pl.when(u < n_units)
            def _():
                pltpu.sync_copy(idx_hbm.at[pl.ds(u * _U, _U)], idx_v)
                copies = []
                for j in range(_U):
                    copies.append(pltpu.async_copy(
                        rm_hbm.at[idx_v.at[j]],
                        rows_v.at[pl.ds(j * _L, _L)],
                        sem,
                    ))
                for c in copies:
                    c.wait()

                @pl.loop(0, _U)
                def _(b):
                    jb = jnp.broadcast_to(b, (16,)).astype(jnp.int32)

                    @pl.loop(0, _L)
                    def _(l):
                        jl = jnp.broadcast_to(l, (16,)).astype(jnp.int32)
                        vals = rows_v[b * _L + l]
                        plsc.store_scatter(tiles_v, [ja, jb, jr, jl], vals)

                pltpu.sync_copy(tiles_v.at[0], out_hbm.at[0, pl.ds(u * _U, _U)])
                pltpu.sync_copy(tiles_v.at[1], out_hbm.at[1, pl.ds(u * _U, _U)])

    rm = relayout(q)
    out4 = gather(rm, idx2d)
    return out4.transpose(1, 3, 0, 2).reshape(n, d)


# speedup vs baseline: 1.4689x; 1.3766x over previous
import functools

import jax
import jax.numpy as jnp
from jax import lax
from jax.experimental import pallas as pl
from jax.experimental.pallas import tpu as pltpu
from jax.experimental.pallas import tpu_sc as plsc

_NC = 2
_NS = 16
_L = 128
_U = 8


def _views(n, d):
    nb = n // _L
    na = d // 8
    return nb, na


def kernel(edge_idxs, table):
    n, d = table.shape
    nb, na = _views(n, d)
    nw = _NC * _NS
    n_units = nb // _U
    max_iters = (n_units + nw - 1) // nw
    mesh = plsc.VectorSubcoreMesh(core_axis_name="c", subcore_axis_name="s")
    cp = pltpu.CompilerParams(use_tc_tiling_on_sc=False,
                              needs_layout_passes=False)

    q = table.reshape(nb, _L, na, 8).transpose(2, 0, 3, 1)
    idx2d = edge_idxs.reshape(nb, _L)

    @functools.partial(
        pl.kernel,
        mesh=mesh,
        out_type=jax.ShapeDtypeStruct((n, d), table.dtype),
        scratch_types=[
            pltpu.VMEM((na, _U, 8, _L), jnp.float32),
            pltpu.VMEM((_U * _L, d), jnp.float32),
            pltpu.SemaphoreType.DMA,
        ],
        compiler_params=cp,
    )
    def relayout(q_hbm, rm_hbm, tin_v, rbuf_v, sem):
        wid = lax.axis_index("s") * _NC + lax.axis_index("c")
        ja = jnp.arange(16, dtype=jnp.int32) // 8
        jr = jnp.arange(16, dtype=jnp.int32) % 8
        z16 = jnp.zeros((16,), jnp.int32)

        @pl.loop(0, max_iters)
        def _(i):
            u = wid + i * nw

            @pl.when(u < n_units)
            def _():
                c0 = pltpu.async_copy(q_hbm.at[0, pl.ds(u * _U, _U)],
                                      tin_v.at[0], sem)
                c1 = pltpu.async_copy(q_hbm.at[1, pl.ds(u * _U, _U)],
                                      tin_v.at[1], sem)
                c0.wait()
                c1.wait()

                @pl.loop(0, _U)
                def _(b):
                    jb = z16 + b

                    @plsc.parallel_loop(0, _L, unroll=8)
                    def _(l):
                        vals = plsc.load_gather(tin_v, [ja, jb, jr, z16 + l])
                        rbuf_v[b * _L + l] = vals

                pltpu.sync_copy(rbuf_v, rm_hbm.at[pl.ds(u * _U * _L, _U * _L)])

    @functools.partial(
        pl.kernel,
        mesh=mesh,
        out_type=jax.ShapeDtypeStruct((na, nb, 8, _L), jnp.float32),
        scratch_types=[
            pltpu.VMEM((_U, _L), jnp.int32),
            pltpu.VMEM((_U * _L, d), jnp.float32),
            pltpu.VMEM((na, _U, 8, _L), jnp.float32),
            pltpu.SemaphoreType.DMA,
        ],
        compiler_params=cp,
    )
    def gather(rm_hbm, idx_hbm, out_hbm, idx_v, rows_v, tiles_v, sem):
        wid = lax.axis_index("s") * _NC + lax.axis_index("c")
        ja = jnp.arange(16, dtype=jnp.int32) // 8
        jr = jnp.arange(16, dtype=jnp.int32) % 8
        z16 = jnp.zeros((16,), jnp.int32)

        @pl.loop(0, max_iters)
        def _(i):
            u = wid + i * nw

            @pl.when(u < n_units)
            def _():
                pltpu.sync_copy(idx_hbm.at[pl.ds(u * _U, _U)], idx_v)
                copies = []
                for j in range(_U):
                    copies.append(pltpu.async_copy(
                        rm_hbm.at[idx_v.at[j]],
                        rows_v.at[pl.ds(j * _L, _L)],
                        sem,
                    ))
                for c in copies:
                    c.wait()

                @pl.loop(0, _U)
                def _(b):
                    jb = z16 + b

                    @plsc.parallel_loop(0, _L, unroll=8)
                    def _(l):
                        vals = rows_v[b * _L + l]
                        plsc.store_scatter(tiles_v, [ja, jb, jr, z16 + l], vals)

                pltpu.sync_copy(tiles_v.at[0], out_hbm.at[0, pl.ds(u * _U, _U)])
                pltpu.sync_copy(tiles_v.at[1], out_hbm.at[1, pl.ds(u * _U, _U)])

    rm = relayout(q)
    out4 = gather(rm, idx2d)
    return out4.transpose(1, 3, 0, 2).reshape(n, d)


# speedup vs baseline: 1.5504x; 1.0555x over previous
import functools

import jax
import jax.numpy as jnp
from jax import lax
from jax.experimental import pallas as pl
from jax.experimental.pallas import tpu as pltpu
from jax.experimental.pallas import tpu_sc as plsc

_NC = 2
_NS = 16
_L = 128
_U = 8
_TILE = _U * 8 * _L


def kernel(edge_idxs, table):
    n, d = table.shape
    nb = n // _L
    na = d // 8
    nw = _NC * _NS
    n_units = nb // _U
    max_iters = (n_units + nw - 1) // nw
    mesh = plsc.VectorSubcoreMesh(core_axis_name="c", subcore_axis_name="s")
    cp = pltpu.CompilerParams(use_tc_tiling_on_sc=False,
                              needs_layout_passes=False)

    q2 = (table.reshape(nb, _L, na, 8).transpose(2, 0, 3, 1)
          .reshape(na, nb * 8 * _L))
    idx2d = edge_idxs.reshape(nb, _L)

    @functools.partial(
        pl.kernel,
        mesh=mesh,
        out_type=jax.ShapeDtypeStruct((n, d), table.dtype),
        scratch_types=[
            pltpu.VMEM((na * _TILE,), jnp.float32),
            pltpu.VMEM((_U * _L, d), jnp.float32),
            pltpu.SemaphoreType.DMA,
        ],
        compiler_params=cp,
    )
    def relayout(q_hbm, rm_hbm, tin_v, rbuf_v, sem):
        wid = lax.axis_index("s") * _NC + lax.axis_index("c")
        jj = jnp.arange(16, dtype=jnp.int32)
        jbase = (jj // 8) * _TILE + (jj % 8) * _L

        @pl.loop(0, max_iters)
        def _(i):
            u = wid + i * nw

            @pl.when(u < n_units)
            def _():
                c0 = pltpu.async_copy(q_hbm.at[0, pl.ds(u * _TILE, _TILE)],
                                      tin_v.at[pl.ds(0, _TILE)], sem)
                c1 = pltpu.async_copy(q_hbm.at[1, pl.ds(u * _TILE, _TILE)],
                                      tin_v.at[pl.ds(_TILE, _TILE)], sem)
                c0.wait()
                c1.wait()

                @pl.loop(0, _U)
                def _(b):
                    jb = jbase + b * (8 * _L)

                    @plsc.parallel_loop(0, _L, unroll=8)
                    def _(l):
                        vals = plsc.load_gather(tin_v, [jb + l])
                        rbuf_v[b * _L + l] = vals

                pltpu.sync_copy(rbuf_v, rm_hbm.at[pl.ds(u * _U * _L, _U * _L)])

    @functools.partial(
        pl.kernel,
        mesh=mesh,
        out_type=jax.ShapeDtypeStruct((na, nb * 8 * _L), jnp.float32),
        scratch_types=[
            pltpu.VMEM((_U, _L), jnp.int32),
            pltpu.VMEM((_U * _L, d), jnp.float32),
            pltpu.VMEM((na * _TILE,), jnp.float32),
            pltpu.SemaphoreType.DMA,
        ],
        compiler_params=cp,
    )
    def gather(rm_hbm, idx_hbm, out_hbm, idx_v, rows_v, tiles_v, sem):
        wid = lax.axis_index("s") * _NC + lax.axis_index("c")
        jj = jnp.arange(16, dtype=jnp.int32)
        jbase = (jj // 8) * _TILE + (jj % 8) * _L

        @pl.loop(0, max_iters)
        def _(i):
            u = wid + i * nw

            @pl.when(u < n_units)
            def _():
                pltpu.sync_copy(idx_hbm.at[pl.ds(u * _U, _U)], idx_v)
                copies = []
                for j in range(_U):
                    copies.append(pltpu.async_copy(
                        rm_hbm.at[idx_v.at[j]],
                        rows_v.at[pl.ds(j * _L, _L)],
                        sem,
                    ))
                for c in copies:
                    c.wait()

                @pl.loop(0, _U)
                def _(b):
                    jb = jbase + b * (8 * _L)

                    @plsc.parallel_loop(0, _L, unroll=8)
                    def _(l):
                        vals = rows_v[b * _L + l]
                        plsc.store_scatter(tiles_v, [jb + l], vals)

                pltpu.sync_copy(tiles_v.at[pl.ds(0, _TILE)],
                                out_hbm.at[0, pl.ds(u * _TILE, _TILE)])
                pltpu.sync_copy(tiles_v.at[pl.ds(_TILE, _TILE)],
                                out_hbm.at[1, pl.ds(u * _TILE, _TILE)])

    rm = relayout(q2)
    out2 = gather(rm, idx2d)
    return (out2.reshape(na, nb, 8, _L).transpose(1, 3, 0, 2)
            .reshape(n, d))


# speedup vs baseline: 2.1464x; 1.3844x over previous
import functools

import jax
import jax.numpy as jnp
from jax import lax
from jax.experimental import pallas as pl
from jax.experimental.pallas import tpu as pltpu
from jax.experimental.pallas import tpu_sc as plsc

_NC = 2
_NS = 16
_L = 128
_U = 8
_R = _U * _L
_TILE = _U * 8 * _L


def kernel(edge_idxs, table):
    n, d = table.shape
    nb = n // _L
    na = d // 8
    nw = _NC * _NS
    n_units = nb // _U
    max_iters = (n_units + nw - 1) // nw
    m2 = ((max_iters + 2) // 2) * 2
    mesh = plsc.VectorSubcoreMesh(core_axis_name="c", subcore_axis_name="s")
    cp = pltpu.CompilerParams(use_tc_tiling_on_sc=False,
                              needs_layout_passes=False)

    q2 = (table.reshape(nb, _L, na, 8).transpose(2, 0, 3, 1)
          .reshape(na, nb * 8 * _L))
    idx2d = edge_idxs.reshape(nb, _L)

    @functools.partial(
        pl.kernel,
        mesh=mesh,
        out_type=jax.ShapeDtypeStruct((n, d), table.dtype),
        scratch_types=[
            pltpu.VMEM((2, na * _TILE), jnp.float32),
            pltpu.VMEM((2, _R, d), jnp.float32),
            pltpu.SemaphoreType.DMA,
            pltpu.SemaphoreType.DMA,
            pltpu.SemaphoreType.DMA,
            pltpu.SemaphoreType.DMA,
        ],
        compiler_params=cp,
    )
    def relayout(q_hbm, rm_hbm, tin_v, rbuf_v, is0, is1, os0, os1):
        wid = lax.axis_index("s") * _NC + lax.axis_index("c")
        jj = jnp.arange(16, dtype=jnp.int32)
        jbase = (jj // 8) * _TILE + (jj % 8) * _L
        isems = (is0, is1)
        osems = (os0, os1)

        def start_in(p, u):
            pltpu.async_copy(q_hbm.at[0, pl.ds(u * _TILE, _TILE)],
                             tin_v.at[p, pl.ds(0, _TILE)], isems[p])
            pltpu.async_copy(q_hbm.at[1, pl.ds(u * _TILE, _TILE)],
                             tin_v.at[p, pl.ds(_TILE, _TILE)], isems[p])

        def wait_in(p):
            pltpu.make_async_copy(q_hbm.at[0, pl.ds(0, _TILE)],
                                  tin_v.at[p, pl.ds(0, _TILE)], isems[p]).wait()
            pltpu.make_async_copy(q_hbm.at[1, pl.ds(0, _TILE)],
                                  tin_v.at[p, pl.ds(_TILE, _TILE)],
                                  isems[p]).wait()

        def start_out(p, u):
            pltpu.async_copy(rbuf_v.at[p], rm_hbm.at[pl.ds(u * _R, _R)],
                             osems[p])

        def wait_out(p):
            pltpu.make_async_copy(rbuf_v.at[p], rm_hbm.at[pl.ds(0, _R)],
                                  osems[p]).wait()

        for p in (0, 1):
            u0 = wid + p * nw

            @pl.when(u0 < n_units)
            def _():
                start_in(p, u0)

        @pl.loop(0, m2, step=2)
        def _(i):
            for p in (0, 1):
                it = i + p
                u = wid + it * nw

                @pl.when(u < n_units)
                def _():
                    wait_in(p)

                    @pl.when(it >= 2)
                    def _():
                        wait_out(p)

                    @pl.loop(0, _U)
                    def _(b):
                        jb = jbase + b * (8 * _L)

                        @plsc.parallel_loop(0, _L, unroll=8)
                        def _(l):
                            vals = plsc.load_gather(tin_v.at[p], [jb + l])
                            rbuf_v[p, b * _L + l] = vals

                    start_out(p, u)
                    un = u + 2 * nw

                    @pl.when(un < n_units)
                    def _():
                        start_in(p, un)

        wait_out(0)
        wait_out(1)

    @functools.partial(
        pl.kernel,
        mesh=mesh,
        out_type=jax.ShapeDtypeStruct((na, nb * 8 * _L), jnp.float32),
        scratch_types=[
            pltpu.VMEM((2, _U, _L), jnp.int32),
            pltpu.VMEM((2, _R, d), jnp.float32),
            pltpu.VMEM((2, na * _TILE), jnp.float32),
            pltpu.SemaphoreType.DMA,
            pltpu.SemaphoreType.DMA,
            pltpu.SemaphoreType.DMA,
            pltpu.SemaphoreType.DMA,
            pltpu.SemaphoreType.DMA,
            pltpu.SemaphoreType.DMA,
        ],
        compiler_params=cp,
    )
    def gather(rm_hbm, idx_hbm, out_hbm, idx_v, rows_v, tiles_v,
               xs0, xs1, gs0, gs1, os0, os1):
        wid = lax.axis_index("s") * _NC + lax.axis_index("c")
        jj = jnp.arange(16, dtype=jnp.int32)
        jbase = (jj // 8) * _TILE + (jj % 8) * _L
        xsems = (xs0, xs1)
        gsems = (gs0, gs1)
        osems = (os0, os1)

        def start_idx(p, u):
            pltpu.async_copy(idx_hbm.at[pl.ds(u * _U, _U)], idx_v.at[p],
                             xsems[p])

        def wait_idx(p):
            pltpu.make_async_copy(idx_hbm.at[pl.ds(0, _U)], idx_v.at[p],
                                  xsems[p]).wait()

        def fire_gathers(p):
            for j in range(_U):
                pltpu.async_copy(rm_hbm.at[idx_v.at[p, j]],
                                 rows_v.at[p, pl.ds(j * _L, _L)], gsems[p])

        def drain_gathers(p):
            for j in range(_U):
                pltpu.make_async_copy(rm_hbm.at[idx_v.at[p, j]],
                                      rows_v.at[p, pl.ds(j * _L, _L)],
                                      gsems[p]).wait()

        def start_out(p, u):
            pltpu.async_copy(tiles_v.at[p, pl.ds(0, _TILE)],
                             out_hbm.at[0, pl.ds(u * _TILE, _TILE)], osems[p])
            pltpu.async_copy(tiles_v.at[p, pl.ds(_TILE, _TILE)],
                             out_hbm.at[1, pl.ds(u * _TILE, _TILE)], osems[p])

        def wait_out(p):
            pltpu.make_async_copy(tiles_v.at[p, pl.ds(0, _TILE)],
                                  out_hbm.at[0, pl.ds(0, _TILE)],
                                  osems[p]).wait()
            pltpu.make_async_copy(tiles_v.at[p, pl.ds(_TILE, _TILE)],
                                  out_hbm.at[1, pl.ds(0, _TILE)],
                                  osems[p]).wait()

        for p in (0, 1):
            u0 = wid + p * nw

            @pl.when(u0 < n_units)
            def _():
                start_idx(p, u0)

        @pl.loop(0, m2, step=2)
        def _(i):
            for p in (0, 1):
                it = i + p
                u = wid + it * nw

                @pl.when(u < n_units)
                def _():
                    wait_idx(p)
                    fire_gathers(p)

                itp = it - 1
                up = wid + itp * nw
                qb = 1 - p

                @pl.when((itp >= 0) & (up < n_units))
                def _():
                    drain_gathers(qb)
                    un = up + 2 * nw

                    @pl.when(un < n_units)
                    def _():
                        start_idx(qb, un)

                    @pl.when(itp >= 2)
                    def _():
                        wait_out(qb)

                    @pl.loop(0, _U)
                    def _(b):
                        jb = jbase + b * (8 * _L)

                        @plsc.parallel_loop(0, _L, unroll=8)
                        def _(l):
                            vals = rows_v[qb, b * _L + l]
                            plsc.store_scatter(tiles_v.at[qb], [jb + l], vals)

                    start_out(qb, up)

        wait_out(0)
        wait_out(1)

    rm = relayout(q2)
    out2 = gather(rm, idx2d)
    return (out2.reshape(na, nb, 8, _L).transpose(1, 3, 0, 2)
            .reshape(n, d))


# speedup vs baseline: 2.1521x; 1.0026x over previous
import functools

import jax
import jax.numpy as jnp
from jax import lax
from jax.experimental import pallas as pl
from jax.experimental.pallas import tpu as pltpu
from jax.experimental.pallas import tpu_sc as plsc

_NC = 2
_NS = 16
_L = 128
_U = 8
_R = _U * _L
_TILE = _U * 8 * _L


def kernel(edge_idxs, table):
    n, d = table.shape
    nb = n // _L
    na = d // 8
    nw = _NC * _NS
    n_units = nb // _U
    max_iters = (n_units + nw - 1) // nw
    m2 = ((max_iters + 2) // 2) * 2
    mesh = plsc.VectorSubcoreMesh(core_axis_name="c", subcore_axis_name="s")
    cp = pltpu.CompilerParams(use_tc_tiling_on_sc=False,
                              needs_layout_passes=False)

    q2 = (table.reshape(nb, _L, na, 8).transpose(2, 0, 3, 1)
          .reshape(na, nb * 8 * _L))

    @functools.partial(
        pl.kernel,
        mesh=mesh,
        out_type=jax.ShapeDtypeStruct((n, d), table.dtype),
        scratch_types=[
            pltpu.VMEM((2, na * _TILE), jnp.float32),
            pltpu.VMEM((2, _R, d), jnp.float32),
            pltpu.SemaphoreType.DMA,
            pltpu.SemaphoreType.DMA,
            pltpu.SemaphoreType.DMA,
            pltpu.SemaphoreType.DMA,
        ],
        compiler_params=cp,
    )
    def relayout(q_hbm, rm_hbm, tin_v, rbuf_v, is0, is1, os0, os1):
        wid = lax.axis_index("s") * _NC + lax.axis_index("c")
        jj = jnp.arange(16, dtype=jnp.int32)
        jbase = (jj // 8) * _TILE + (jj % 8) * _L
        isems = (is0, is1)
        osems = (os0, os1)

        def start_in(p, u):
            pltpu.async_copy(q_hbm.at[0, pl.ds(u * _TILE, _TILE)],
                             tin_v.at[p, pl.ds(0, _TILE)], isems[p])
            pltpu.async_copy(q_hbm.at[1, pl.ds(u * _TILE, _TILE)],
                             tin_v.at[p, pl.ds(_TILE, _TILE)], isems[p])

        def wait_in(p):
            pltpu.make_async_copy(q_hbm.at[0, pl.ds(0, _TILE)],
                                  tin_v.at[p, pl.ds(0, _TILE)], isems[p]).wait()
            pltpu.make_async_copy(q_hbm.at[1, pl.ds(0, _TILE)],
                                  tin_v.at[p, pl.ds(_TILE, _TILE)],
                                  isems[p]).wait()

        def start_out(p, u):
            pltpu.async_copy(rbuf_v.at[p], rm_hbm.at[pl.ds(u * _R, _R)],
                             osems[p])

        def wait_out(p):
            pltpu.make_async_copy(rbuf_v.at[p], rm_hbm.at[pl.ds(0, _R)],
                                  osems[p]).wait()

        for p in (0, 1):
            u0 = wid + p * nw

            @pl.when(u0 < n_units)
            def _():
                start_in(p, u0)

        @pl.loop(0, m2, step=2)
        def _(i):
            for p in (0, 1):
                it = i + p
                u = wid + it * nw

                @pl.when(u < n_units)
                def _():
                    wait_in(p)

                    @pl.when(it >= 2)
                    def _():
                        wait_out(p)

                    @pl.loop(0, _U)
                    def _(b):
                        jb = jbase + b * (8 * _L)

                        @plsc.parallel_loop(0, _L, unroll=8)
                        def _(l):
                            vals = plsc.load_gather(tin_v.at[p], [jb + l])
                            rbuf_v[p, b * _L + l] = vals

                    start_out(p, u)
                    un = u + 2 * nw

                    @pl.when(un < n_units)
                    def _():
                        start_in(p, un)

        wait_out(0)
        wait_out(1)

    @functools.partial(
        pl.kernel,
        mesh=mesh,
        out_type=jax.ShapeDtypeStruct((na, nb * 8 * _L), jnp.float32),
        scratch_types=[
            pltpu.VMEM((2, _R), jnp.int32),
            pltpu.VMEM((2, _R, d), jnp.float32),
            pltpu.VMEM((2, na * _TILE), jnp.float32),
            pltpu.SemaphoreType.DMA,
            pltpu.SemaphoreType.DMA,
            pltpu.SemaphoreType.DMA,
            pltpu.SemaphoreType.DMA,
            pltpu.SemaphoreType.DMA,
            pltpu.SemaphoreType.DMA,
        ],
        compiler_params=cp,
    )
    def gather(rm_hbm, idx_hbm, out_hbm, idx_v, rows_v, tiles_v,
               xs0, xs1, gs0, gs1, os0, os1):
        wid = lax.axis_index("s") * _NC + lax.axis_index("c")
        jj = jnp.arange(16, dtype=jnp.int32)
        jbase = (jj // 8) * _TILE + (jj % 8) * _L
        xsems = (xs0, xs1)
        gsems = (gs0, gs1)
        osems = (os0, os1)

        def start_idx(p, u):
            pltpu.async_copy(idx_hbm.at[pl.ds(u * _R, _R)], idx_v.at[p],
                             xsems[p])

        def wait_idx(p):
            pltpu.make_async_copy(idx_hbm.at[pl.ds(0, _R)], idx_v.at[p],
                                  xsems[p]).wait()

        def fire_gathers(p):
            pltpu.async_copy(rm_hbm.at[idx_v.at[p]], rows_v.at[p], gsems[p])

        def drain_gathers(p):
            pltpu.make_async_copy(rm_hbm.at[idx_v.at[p]], rows_v.at[p],
                                  gsems[p]).wait()

        def start_out(p, u):
            pltpu.async_copy(tiles_v.at[p, pl.ds(0, _TILE)],
                             out_hbm.at[0, pl.ds(u * _TILE, _TILE)], osems[p])
            pltpu.async_copy(tiles_v.at[p, pl.ds(_TILE, _TILE)],
                             out_hbm.at[1, pl.ds(u * _TILE, _TILE)], osems[p])

        def wait_out(p):
            pltpu.make_async_copy(tiles_v.at[p, pl.ds(0, _TILE)],
                                  out_hbm.at[0, pl.ds(0, _TILE)],
                                  osems[p]).wait()
            pltpu.make_async_copy(tiles_v.at[p, pl.ds(_TILE, _TILE)],
                                  out_hbm.at[1, pl.ds(0, _TILE)],
                                  osems[p]).wait()

        for p in (0, 1):
            u0 = wid + p * nw

            @pl.when(u0 < n_units)
            def _():
                start_idx(p, u0)

        @pl.loop(0, m2, step=2)
        def _(i):
            for p in (0, 1):
                it = i + p
                u = wid + it * nw

                @pl.when(u < n_units)
                def _():
                    wait_idx(p)
                    fire_gathers(p)

                itp = it - 1
                up = wid + itp * nw
                qb = 1 - p

                @pl.when((itp >= 0) & (up < n_units))
                def _():
                    drain_gathers(qb)
                    un = up + 2 * nw

                    @pl.when(un < n_units)
                    def _():
                        start_idx(qb, un)

                    @pl.when(itp >= 2)
                    def _():
                        wait_out(qb)

                    @pl.loop(0, _U)
                    def _(b):
                        jb = jbase + b * (8 * _L)

                        @plsc.parallel_loop(0, _L, unroll=8)
                        def _(l):
                            vals = rows_v[qb, b * _L + l]
                            plsc.store_scatter(tiles_v.at[qb], [jb + l], vals)

                    start_out(qb, up)

        wait_out(0)
        wait_out(1)

    rm = relayout(q2)
    out2 = gather(rm, edge_idxs)
    return (out2.reshape(na, nb, 8, _L).transpose(1, 3, 0, 2)
            .reshape(n, d))
